# baseline (speedup 1.0000x reference)
import functools

import jax
import jax.numpy as jnp
from jax import lax
from jax.experimental import pallas as pl
from jax.experimental.pallas import tpu as pltpu

_K = 128
_LANES = 128
_G = 1024


def _ce_sort(v, k, d, row0=0):
    n, lanes = v.shape
    dd = 8 * d
    r_pairs = n // (2 * dd)
    x = v.reshape(r_pairs, 2, dd, lanes)
    a, b = x[:, 0], x[:, 1]
    mx = jnp.maximum(a, b)
    mn = jnp.minimum(a, b)
    nblk = n >> (k + 4)
    if nblk == 0:
        if (row0 >> (k + 3)) & 1:
            top, bot = mn, mx
        else:
            top, bot = mx, mn
    else:
        p = (1 << (k - 1)) // d
        mx5 = mx.reshape(nblk, 2, p, dd, lanes)
        mn5 = mn.reshape(nblk, 2, p, dd, lanes)
        top = jnp.concatenate([mx5[:, 0:1], mn5[:, 1:2]], axis=1)
        bot = jnp.concatenate([mn5[:, 0:1], mx5[:, 1:2]], axis=1)
        top = top.reshape(r_pairs, dd, lanes)
        bot = bot.reshape(r_pairs, dd, lanes)
    return jnp.stack([top, bot], axis=1).reshape(n, lanes)


def _ce_clean(v, d):
    n, lanes = v.shape
    dd = 8 * d
    r_pairs = n // (2 * dd)
    x = v.reshape(r_pairs, 2, dd, lanes)
    a, b = x[:, 0], x[:, 1]
    mx = jnp.maximum(a, b)
    mn = jnp.minimum(a, b)
    return jnp.stack([mx, mn], axis=1).reshape(n, lanes)


def _sort_group(v):
    segs = []
    for si in range(v.shape[0] // 128):
        w = v[si * 128:(si + 1) * 128]
        for k in range(1, 5):
            d = 1 << (k - 1)
            while d:
                w = _ce_sort(w, k, d, row0=si * 128)
                d >>= 1
        segs.append(w)
    v = jnp.concatenate(segs, axis=0)
    for k in range(5, 8):
        d = 1 << (k - 1)
        while d >= 16:
            v = _ce_sort(v, k, d)
            d >>= 1
        segs = []
        for si in range(v.shape[0] // 128):
            w = v[si * 128:(si + 1) * 128]
            ds_ = 8
            while ds_:
                w = _ce_sort(w, k, ds_, row0=si * 128)
                ds_ >>= 1
            segs.append(w)
        v = jnp.concatenate(segs, axis=0)
    return v


def _cleanup(v):
    for d in (64, 32, 16):
        v = _ce_clean(v, d)
    segs = []
    for si in range(v.shape[0] // 128):
        w = v[si * 128:(si + 1) * 128]
        for d in (8, 4, 2, 1):
            w = _ce_clean(w, d)
        segs.append(w)
    return jnp.concatenate(segs, axis=0)


def _blockrev(v):
    n = v.shape[0]
    return jnp.concatenate(
        [v[i * 8:(i + 1) * 8] for i in reversed(range(n // 8))], axis=0)


def _body(x_ref, o_ref, s_ref, *, s):
    ngroups = s // _G

    def p1(g, carry):
        base = pl.multiple_of(g * _G, _G)
        v = x_ref[0, pl.ds(base, _G), :]
        s_ref[pl.ds(base, _G), :] = _sort_group(v)
        return carry

    lax.fori_loop(0, ngroups, p1, 0)

    def merge_groups(ga, gb):
        a = s_ref[pl.ds(pl.multiple_of(ga * _G, _G), _G), :]
        b = s_ref[pl.ds(pl.multiple_of(gb * _G, _G), _G), :]
        m = _cleanup(jnp.maximum(a, _blockrev(b)))
        s_ref[pl.ds(pl.multiple_of(ga * _G, _G), _G), :] = m

    def l1(u, carry):
        merge_groups(2 * u, 2 * u + 1)
        return carry

    lax.fori_loop(0, ngroups // 2, l1, 0)
    merge_groups(0, 2)
    merge_groups(4, 6)
    merge_groups(0, 4)

    v = s_ref[0:_G, :]
    for shift in (1, 2, 4):
        w = _blockrev(v)
        w = jnp.concatenate([w[shift:], w[:shift]], axis=0)
        v = _cleanup(jnp.maximum(v, w))

    o_ref[0] = v.reshape(_K, 8, v.shape[-1])[:, 0, :]


def kernel(inputs):
    b, s, d = inputs.shape
    grid = (b, d // _LANES)
    out = pl.pallas_call(
        functools.partial(_body, s=s),
        grid=grid,
        in_specs=[pl.BlockSpec((1, s, _LANES), lambda bi, j: (bi, 0, j))],
        out_specs=pl.BlockSpec((1, _K, _LANES), lambda bi, j: (bi, 0, j)),
        out_shape=jax.ShapeDtypeStruct((b, _K, d), jnp.float32),
        scratch_shapes=[pltpu.VMEM((s, _LANES), jnp.float32)],
        compiler_params=pltpu.CompilerParams(
            dimension_semantics=("parallel", "parallel")),
    )(inputs)
    return out


# speedup vs baseline: 1.0381x; 1.0381x over previous
import functools

import jax
import jax.numpy as jnp
from jax import lax
from jax.experimental import pallas as pl
from jax.experimental.pallas import tpu as pltpu

_K = 128
_LANES = 128
_G = 1024


def _ce_sort(v, k, d, row0=0):
    n, lanes = v.shape
    dd = 8 * d
    r_pairs = n // (2 * dd)
    x = v.reshape(r_pairs, 2, dd, lanes)
    a, b = x[:, 0], x[:, 1]
    mx = jnp.maximum(a, b)
    mn = jnp.minimum(a, b)
    nblk = n >> (k + 4)
    if nblk == 0:
        if (row0 >> (k + 3)) & 1:
            top, bot = mn, mx
        else:
            top, bot = mx, mn
    else:
        p = (1 << (k - 1)) // d
        mx5 = mx.reshape(nblk, 2, p, dd, lanes)
        mn5 = mn.reshape(nblk, 2, p, dd, lanes)
        top = jnp.concatenate([mx5[:, 0:1], mn5[:, 1:2]], axis=1)
        bot = jnp.concatenate([mn5[:, 0:1], mx5[:, 1:2]], axis=1)
        top = top.reshape(r_pairs, dd, lanes)
        bot = bot.reshape(r_pairs, dd, lanes)
    return jnp.stack([top, bot], axis=1).reshape(n, lanes)


def _ce_clean(v, d):
    n, lanes = v.shape
    dd = 8 * d
    r_pairs = n // (2 * dd)
    x = v.reshape(r_pairs, 2, dd, lanes)
    a, b = x[:, 0], x[:, 1]
    mx = jnp.maximum(a, b)
    mn = jnp.minimum(a, b)
    return jnp.stack([mx, mn], axis=1).reshape(n, lanes)


def _sort_group(v):
    segs = []
    for si in range(v.shape[0] // 128):
        w = v[si * 128:(si + 1) * 128]
        for k in range(1, 5):
            d = 1 << (k - 1)
            while d:
                w = _ce_sort(w, k, d, row0=si * 128)
                d >>= 1
        segs.append(w)
    v = jnp.concatenate(segs, axis=0)
    for k in range(5, 8):
        d = 1 << (k - 1)
        while d >= 32:
            v = _ce_sort(v, k, d)
            d >>= 1
        segs = []
        for si in range(v.shape[0] // 256):
            w = v[si * 256:(si + 1) * 256]
            ds_ = 16
            while ds_:
                w = _ce_sort(w, k, ds_, row0=si * 256)
                ds_ >>= 1
            segs.append(w)
        v = jnp.concatenate(segs, axis=0)
    return v


def _cleanup(v):
    for d in (64, 32):
        v = _ce_clean(v, d)
    segs = []
    for si in range(v.shape[0] // 256):
        w = v[si * 256:(si + 1) * 256]
        for d in (16, 8, 4, 2, 1):
            w = _ce_clean(w, d)
        segs.append(w)
    return jnp.concatenate(segs, axis=0)


def _blockrev(v):
    n = v.shape[0]
    return jnp.concatenate(
        [v[i * 8:(i + 1) * 8] for i in reversed(range(n // 8))], axis=0)


def _body(x_ref, o_ref, s_ref, *, s):
    ngroups = s // _G

    def p1(g, carry):
        base = pl.multiple_of(g * _G, _G)
        v = x_ref[0, pl.ds(base, _G), :]
        s_ref[pl.ds(base, _G), :] = _sort_group(v)
        return carry

    lax.fori_loop(0, ngroups, p1, 0)

    def merge_groups(ga, gb):
        a = s_ref[pl.ds(pl.multiple_of(ga * _G, _G), _G), :]
        b = s_ref[pl.ds(pl.multiple_of(gb * _G, _G), _G), :]
        m = _cleanup(jnp.maximum(a, _blockrev(b)))
        s_ref[pl.ds(pl.multiple_of(ga * _G, _G), _G), :] = m

    def l1(u, carry):
        merge_groups(2 * u, 2 * u + 1)
        return carry

    lax.fori_loop(0, ngroups // 2, l1, 0)
    merge_groups(0, 2)
    merge_groups(4, 6)
    merge_groups(0, 4)

    v = s_ref[0:_G, :]
    for shift in (1, 2, 4):
        w = _blockrev(v)
        w = jnp.concatenate([w[shift:], w[:shift]], axis=0)
        v = _cleanup(jnp.maximum(v, w))

    o_ref[0] = v.reshape(_K, 8, v.shape[-1])[:, 0, :]


def kernel(inputs):
    b, s, d = inputs.shape
    grid = (b, d // _LANES)
    out = pl.pallas_call(
        functools.partial(_body, s=s),
        grid=grid,
        in_specs=[pl.BlockSpec((1, s, _LANES), lambda bi, j: (bi, 0, j))],
        out_specs=pl.BlockSpec((1, _K, _LANES), lambda bi, j: (bi, 0, j)),
        out_shape=jax.ShapeDtypeStruct((b, _K, d), jnp.float32),
        scratch_shapes=[pltpu.VMEM((s, _LANES), jnp.float32)],
        compiler_params=pltpu.CompilerParams(
            dimension_semantics=("parallel", "parallel")),
    )(inputs)
    return out
